# baseline (speedup 1.0000x reference)
import functools

import jax
import jax.numpy as jnp
from jax import lax
from jax.experimental import pallas as pl
from jax.experimental.pallas import tpu as pltpu
from jax.experimental.pallas import tpu_sc as plsc

B = 16384
F = 100
D = 128
V = 100000

_NC = 2
_NS = 16
_NW = _NC * _NS
_N = B * F
_PER_W = _N // _NW
_CHUNK = 6400
_NCHUNK = _PER_W // _CHUNK


_ROWS_W = B // _NW
_ROWS_C = 64
_NRCH = _ROWS_W // _ROWS_C
_FPAD = 128


def _gather_body(w_hbm, idx_hbm, out_hbm, table_v, idx_v, out_v):
    wid = lax.axis_index("s") * _NC + lax.axis_index("c")
    row0 = wid * _ROWS_W
    pltpu.sync_copy(w_hbm, table_v)

    lanes = lax.iota(jnp.int32, 16)
    tail_rows = lanes // 4
    tail_cols = 96 + lanes % 4

    def chunk_body(j, carry):
        r = pl.multiple_of(row0 + j * _ROWS_C, _ROWS_C)
        pltpu.sync_copy(idx_hbm.at[pl.ds(r, _ROWS_C)], idx_v)

        def inner(i, c):
            row = i // 6
            k = i % 6
            sl = pl.ds(pl.multiple_of(k * 16, 16), 16)
            out_v[row, sl] = plsc.load_gather(table_v, [idx_v[row, sl]])
            return c

        lax.fori_loop(0, _ROWS_C * 6, inner, 0, unroll=6)

        def tail(t, c):
            rows = tail_rows + 4 * t
            tidx = plsc.load_gather(idx_v, [rows, tail_cols])
            vals = plsc.load_gather(table_v, [tidx])
            plsc.store_scatter(out_v, [rows, tail_cols], vals)
            return c

        lax.fori_loop(0, _ROWS_C // 4, tail, 0, unroll=4)
        pltpu.sync_copy(out_v, out_hbm.at[pl.ds(r, _ROWS_C)])
        return carry

    lax.fori_loop(0, _NRCH, chunk_body, 0)


_sc_gather = pl.kernel(
    _gather_body,
    out_type=jax.ShapeDtypeStruct((B, F), jnp.float32),
    mesh=plsc.VectorSubcoreMesh(core_axis_name="c", subcore_axis_name="s"),
    scratch_types=[
        pltpu.VMEM((V,), jnp.float32),
        pltpu.VMEM((_ROWS_C, F), jnp.int32),
        pltpu.VMEM((_ROWS_C, F), jnp.float32),
    ],
    compiler_params=pltpu.CompilerParams(
        needs_layout_passes=False, use_tc_tiling_on_sc=False),
)


_BB = 2048
_FB = 10
_NB = B // _BB
_NF = F // _FB


def _second_body(e_ref, o_ref, s_acc, sq_acc):
    j = pl.program_id(1)
    e = e_ref[...]
    s = jnp.sum(e, axis=0)
    sq = jnp.sum(e * e, axis=0)

    @pl.when(j == 0)
    def _init():
        s_acc[...] = s
        sq_acc[...] = sq

    @pl.when(j != 0)
    def _accum():
        s_acc[...] += s
        sq_acc[...] += sq

    @pl.when(j == _NF - 1)
    def _fin():
        st = s_acc[...]
        o_ref[...] = 0.5 * (st * st - sq_acc[...])


_second = pl.pallas_call(
    _second_body,
    grid=(_NB, _NF),
    in_specs=[pl.BlockSpec((_FB, _BB, D), lambda i, j: (j, i, 0))],
    out_specs=pl.BlockSpec((_BB, D), lambda i, j: (i, 0)),
    out_shape=jax.ShapeDtypeStruct((B, D), jnp.float32),
    scratch_shapes=[
        pltpu.VMEM((_BB, D), jnp.float32),
        pltpu.VMEM((_BB, D), jnp.float32),
    ],
)


def kernel(sparse_inputs, embed_inputs, w):
    first = _sc_gather(w.reshape(-1), sparse_inputs)
    second = _second(jnp.transpose(embed_inputs, (1, 0, 2)))
    return jnp.concatenate([first, second], axis=-1)


# speedup vs baseline: 1.0546x; 1.0546x over previous
import jax
import jax.numpy as jnp
from jax import lax
from jax.experimental import pallas as pl
from jax.experimental.pallas import tpu as pltpu
from jax.experimental.pallas import tpu_sc as plsc

B = 16384
F = 100
D = 128
V = 100000

_NC = 2
_NS = 16
_NW = _NC * _NS
_COLS_W = B // _NW
_COLS_C = 64
_NCCH = _COLS_W // _COLS_C


def _gather_body(w_hbm, idx_hbm, out_hbm, table_v, idx_v, out_v):
    wid = lax.axis_index("s") * _NC + lax.axis_index("c")
    col0 = wid * _COLS_W
    pltpu.sync_copy(w_hbm, table_v)

    lanes = lax.iota(jnp.int32, 16)
    tail_f = 96 + lanes % 4
    tail_c = lanes // 4

    def chunk_body(j, carry):
        r = pl.multiple_of(col0 + j * _COLS_C, _COLS_C)
        pltpu.sync_copy(idx_hbm.at[pl.ds(r, _COLS_C)], idx_v)

        def inner(i, cr):
            c = i // 6
            k = i % 6
            sl = pl.ds(pl.multiple_of(k * 16, 16), 16)
            vals = plsc.load_gather(table_v, [idx_v[c, sl]])
            plsc.store_scatter(
                out_v, [k * 16 + lanes, jnp.full((16,), c, jnp.int32)], vals)
            return cr

        lax.fori_loop(0, _COLS_C * 6, inner, 0, unroll=6)

        def tail(t, cr):
            cols = tail_c + 4 * t
            tidx = plsc.load_gather(idx_v, [cols, tail_f])
            vals = plsc.load_gather(table_v, [tidx])
            plsc.store_scatter(out_v, [tail_f, cols], vals)
            return cr

        lax.fori_loop(0, _COLS_C // 4, tail, 0, unroll=4)
        pltpu.sync_copy(out_v, out_hbm.at[:, pl.ds(r, _COLS_C)])
        return carry

    lax.fori_loop(0, _NCCH, chunk_body, 0)


_sc_gather = pl.kernel(
    _gather_body,
    out_type=jax.ShapeDtypeStruct((F, B), jnp.float32),
    mesh=plsc.VectorSubcoreMesh(core_axis_name="c", subcore_axis_name="s"),
    scratch_types=[
        pltpu.VMEM((V,), jnp.float32),
        pltpu.VMEM((_COLS_C, F), jnp.int32),
        pltpu.VMEM((F, _COLS_C), jnp.float32),
    ],
    compiler_params=pltpu.CompilerParams(
        needs_layout_passes=False, use_tc_tiling_on_sc=False),
)


_BB = 2048
_FB = 10
_NB = B // _BB
_NF = F // _FB


def _second_body(e_ref, o_ref, s_acc, sq_acc):
    j = pl.program_id(1)
    e = e_ref[...]
    s = jnp.sum(e, axis=0)
    sq = jnp.sum(e * e, axis=0)

    @pl.when(j == 0)
    def _init():
        s_acc[...] = s
        sq_acc[...] = sq

    @pl.when(j != 0)
    def _accum():
        s_acc[...] += s
        sq_acc[...] += sq

    @pl.when(j == _NF - 1)
    def _fin():
        st = s_acc[...]
        res = 0.5 * (st * st - sq_acc[...])
        ident = jnp.eye(D, dtype=jnp.float32)
        o_ref[...] = lax.dot_general(
            ident, res, (((1,), (1,)), ((), ())),
            preferred_element_type=jnp.float32)


_second = pl.pallas_call(
    _second_body,
    grid=(_NB, _NF),
    in_specs=[pl.BlockSpec((_FB, _BB, D), lambda i, j: (j, i, 0))],
    out_specs=pl.BlockSpec((D, _BB), lambda i, j: (0, i)),
    out_shape=jax.ShapeDtypeStruct((D, B), jnp.float32),
    scratch_shapes=[
        pltpu.VMEM((_BB, D), jnp.float32),
        pltpu.VMEM((_BB, D), jnp.float32),
    ],
)


def kernel(sparse_inputs, embed_inputs, w):
    first_t = _sc_gather(w.reshape(-1), sparse_inputs)
    second_t = _second(jnp.transpose(embed_inputs, (1, 0, 2)))
    out_t = jnp.concatenate([first_t, second_t], axis=0)
    return out_t.T
